# Initial kernel scaffold; baseline (speedup 1.0000x reference)
import jax
import jax.numpy as jnp
from jax.experimental import pallas as pl


def kernel(embeds, edge_index, anchorset_id, dists_array, Wh, bh, qTrans, kTrans, vTrans):
    raise NotImplementedError("write your pallas kernel here")


# speedup vs baseline: 5.5141x; 5.5141x over previous
import functools

import jax
import jax.numpy as jnp
from jax import lax
from jax.experimental import pallas as pl
from jax.experimental.pallas import tpu as pltpu
from jax.experimental.pallas import tpu_sc as plsc

_N = 10000
_EMB = 32
_ANCH = 32
_HEADS = 4
_DH = 8
_E0 = 640000
_ADD = int(_E0 * 0.01)
_ETOT = 2 * _ADD + _N + _E0
_L = 16
_NW = 32
_CHUNK = 128
_NCH = -(-_ETOT // (_NW * _CHUNK))
_PER_TILE = _NCH * _CHUNK
_EPAD = _NW * _PER_TILE
_NPAD = _N + 8
_HPAD = 8


def _qk_body(emb_ref, dst_ref, se_ref, w1_ref, w2_ref, bh_ref, qt_ref,
             kt_ref, q_ref, k_ref):
    f32 = jnp.float32
    sw = jnp.dot(se_ref[...], w1_ref[...], preferred_element_type=f32)
    pos = (jnp.dot(dst_ref[...], sw, preferred_element_type=f32) * (1.0 / _ANCH)
           + jnp.dot(emb_ref[...], w2_ref[...], preferred_element_type=f32)
           + bh_ref[...])
    q_ref[...] = jnp.dot(pos, qt_ref[...], preferred_element_type=f32)
    k_ref[...] = jnp.dot(pos, kt_ref[...], preferred_element_type=f32)


_mesh = plsc.VectorSubcoreMesh(core_axis_name="c", subcore_axis_name="s")


def _edge_attention_body(q_hbm, k_hbm, rows_hbm, cols_hbm, z_hbm,
                         exp_hbm, na_hbm, nb_hbm,
                         rv, cv, qv, kv, vals, evals, nsh, sem1, sem2):
    c = lax.axis_index("c")
    s = lax.axis_index("s")
    wid = s * 2 + c
    pltpu.sync_copy(z_hbm.at[pl.ds(0, _CHUNK)], vals)

    @pl.when(s == 0)
    def _():
        pltpu.sync_copy(z_hbm, nsh)

    plsc.subcore_barrier()
    base = wid * _PER_TILE

    def chunk_body(i, carry):
        off = base + i * _CHUNK
        pltpu.sync_copy(rows_hbm.at[pl.ds(off, _CHUNK)], rv)
        pltpu.sync_copy(cols_hbm.at[pl.ds(off, _CHUNK)], cv)
        cp1 = pltpu.async_copy(q_hbm.at[rv], qv, sem1)
        cp2 = pltpu.async_copy(k_hbm.at[cv], kv, sem2)
        cp1.wait()
        cp2.wait()
        for g in range(_CHUNK // _L):
            ei = lax.iota(jnp.int32, _L) + (g * _L)
            for h in range(_HEADS):
                acc = None
                for d in range(_DH):
                    ci = jnp.full((_L,), h * _DH + d, jnp.int32)
                    qc = plsc.load_gather(qv, [ei, ci])
                    kc = plsc.load_gather(kv, [ei, ci])
                    acc = qc * kc if acc is None else acc + qc * kc
                att = jnp.minimum(jnp.maximum(acc, -10.0), 10.0)
                ex = jnp.exp(att)
                hs = jnp.full((_L,), h, jnp.int32)
                plsc.store_scatter(vals, [ei, hs], ex)
                plsc.store_scatter(evals, [ei, hs], ex)
        pltpu.sync_copy(evals, exp_hbm.at[pl.ds(off, _CHUNK)])
        pltpu.sync_copy(vals, nsh.at[rv], add=True)
        return carry

    lax.fori_loop(0, _NCH, chunk_body, 0)
    plsc.subcore_barrier()

    @pl.when(jnp.logical_and(s == 0, c == 0))
    def _():
        pltpu.sync_copy(nsh, na_hbm)

    @pl.when(jnp.logical_and(s == 0, c == 1))
    def _():
        pltpu.sync_copy(nsh, nb_hbm)


_edge_attention = functools.partial(
    pl.kernel,
    out_type=[
        jax.ShapeDtypeStruct((_EPAD, _HEADS), jnp.float32),
        jax.ShapeDtypeStruct((_NPAD, _HPAD), jnp.float32),
        jax.ShapeDtypeStruct((_NPAD, _HPAD), jnp.float32),
    ],
    scratch_types=[
        pltpu.VMEM((_CHUNK,), jnp.int32),
        pltpu.VMEM((_CHUNK,), jnp.int32),
        pltpu.VMEM((_CHUNK, _EMB), jnp.float32),
        pltpu.VMEM((_CHUNK, _EMB), jnp.float32),
        pltpu.VMEM((_CHUNK, _HPAD), jnp.float32),
        pltpu.VMEM((_CHUNK, _HEADS), jnp.float32),
        pltpu.VMEM_SHARED((_NPAD, _HPAD), jnp.float32),
        pltpu.SemaphoreType.DMA,
        pltpu.SemaphoreType.DMA,
    ],
    mesh=_mesh,
    compiler_params=pltpu.CompilerParams(
        needs_layout_passes=False, use_tc_tiling_on_sc=False),
)(_edge_attention_body)


def _normalize_body(rows_hbm, exp_hbm, na_hbm, nb_hbm, out_hbm,
                    rv, ev, nav, nbv, av, sem1, sem2):
    c = lax.axis_index("c")
    s = lax.axis_index("s")
    wid = s * 2 + c
    base = wid * _PER_TILE

    def chunk_body(i, carry):
        off = base + i * _CHUNK
        pltpu.sync_copy(rows_hbm.at[pl.ds(off, _CHUNK)], rv)
        pltpu.sync_copy(exp_hbm.at[pl.ds(off, _CHUNK)], ev)
        cp1 = pltpu.async_copy(na_hbm.at[rv], nav, sem1)
        cp2 = pltpu.async_copy(nb_hbm.at[rv], nbv, sem2)
        cp1.wait()
        cp2.wait()
        for g in range(_CHUNK // _L):
            ei = lax.iota(jnp.int32, _L) + (g * _L)
            acc = jnp.zeros((_L,), jnp.float32)
            for h in range(_HEADS):
                hs = jnp.full((_L,), h, jnp.int32)
                eh = plsc.load_gather(ev, [ei, hs])
                nh = plsc.load_gather(nav, [ei, hs]) + plsc.load_gather(nbv, [ei, hs])
                acc = acc + eh / (nh + 1e-8)
            av[pl.ds(g * _L, _L)] = acc
        pltpu.sync_copy(av, out_hbm.at[pl.ds(off, _CHUNK)])
        return carry

    lax.fori_loop(0, _NCH, chunk_body, 0)


_normalize = functools.partial(
    pl.kernel,
    out_type=jax.ShapeDtypeStruct((_EPAD,), jnp.float32),
    scratch_types=[
        pltpu.VMEM((_CHUNK,), jnp.int32),
        pltpu.VMEM((_CHUNK, _HEADS), jnp.float32),
        pltpu.VMEM((_CHUNK, _HPAD), jnp.float32),
        pltpu.VMEM((_CHUNK, _HPAD), jnp.float32),
        pltpu.VMEM((_CHUNK,), jnp.float32),
        pltpu.SemaphoreType.DMA,
        pltpu.SemaphoreType.DMA,
    ],
    mesh=_mesh,
    compiler_params=pltpu.CompilerParams(
        needs_layout_passes=False, use_tc_tiling_on_sc=False),
)(_normalize_body)


def kernel(embeds, edge_index, anchorset_id, dists_array, Wh, bh, qTrans,
           kTrans, vTrans):
    del vTrans
    f32 = jnp.float32
    set_emb = jnp.take(embeds, anchorset_id, axis=0)
    w1 = Wh[:_EMB]
    w2 = Wh[_EMB:]
    emb_p = jnp.pad(embeds, ((0, _NPAD - _N), (0, 0)))
    dst_p = jnp.pad(dists_array, ((0, _NPAD - _N), (0, 0)))
    q_tab, k_tab = pl.pallas_call(
        _qk_body,
        out_shape=[jax.ShapeDtypeStruct((_NPAD, _EMB), f32)] * 2,
    )(emb_p, dst_p, set_emb, w1, w2, bh.reshape(1, _EMB), qTrans, kTrans)

    rows = edge_index[0]
    cols = edge_index[1]
    ka, kb = jax.random.split(jax.random.key(1))
    tr = rows[jax.random.randint(ka, (_ADD,), 0, _E0)]
    tc = cols[jax.random.randint(kb, (_ADD,), 0, _E0)]
    loop = jnp.arange(_N, dtype=rows.dtype)
    new_rows = jnp.concatenate([tr, tc, loop, rows])
    new_cols = jnp.concatenate([tc, tr, loop, cols])
    rows_p = jnp.pad(new_rows, (0, _EPAD - _ETOT), constant_values=_N)
    cols_p = jnp.pad(new_cols, (0, _EPAD - _ETOT), constant_values=_N)
    z = jnp.zeros((_NPAD, _HPAD), f32)

    exp_e, na, nb = _edge_attention(q_tab, k_tab, rows_p, cols_p, z)
    att = _normalize(rows_p, exp_e, na, nb)
    return att[:_ETOT], new_rows, new_cols
